# baseline (speedup 1.0000x reference)
import jax
import jax.numpy as jnp
from jax.experimental import pallas as pl

N_NODES_K = 10000
N_EDGES_K = 320000
N_LG_K = 640000
N_GRAPHS_K = 128
DK = 128
N_ITER_K = 3


def _bn_k(x, g, b):
    m = jnp.mean(x, axis=0)
    v = jnp.var(x, axis=0)
    return (x - m) / jnp.sqrt(v + 1e-5) * g + b


def _prelu_k(x, w):
    return jnp.where(x >= 0, x, w * x)


def _linear_block_k(x, p):
    x1 = _bn_k(x, p['g1'], p['b1']) @ p['W1'] + p['c1']
    x2 = _prelu_k(_bn_k(x1, p['g2'], p['b2']), p['p2']) @ p['W2'] + p['c2']
    x3 = _prelu_k(_bn_k(x2, p['g3'], p['b3']), p['p3']) @ p['W3'] + p['c3']
    x = (x3 + x1) / 2.0
    x4 = _prelu_k(_bn_k(x, p['g4'], p['b4']), p['p4']) @ p['W4'] + p['c4']
    x = (x4 + x) / 2.0
    return _prelu_k(_bn_k(x, p['g5'], p['b5']), p['p5']) @ p['W5'] + p['c5']


def _copy_kernel(x_ref, o_ref):
    o_ref[...] = x_ref[...]


def kernel(x, edge_attr, params, edge_index, line_graph_edge_index, edge_index_batch):
    lg = line_graph_edge_index
    batch = edge_index_batch
    eu = x @ params['Wu']
    ev = x @ params['Wv']
    euv = edge_attr @ params['We']
    ea = (eu[edge_index[0]] + ev[edge_index[1]] + euv) / 3.0
    out = ea
    out_list = []
    gout_list = []
    for _ in range(N_ITER_K):
        agg = jax.ops.segment_sum(out[lg[0]], lg[1], num_segments=N_EDGES_K)
        out = ea + agg
        conv_agg = jax.ops.segment_sum(out[lg[0]], lg[1], num_segments=N_EDGES_K)
        xc = conv_agg @ params['Wrel'] + params['crel'] + out @ params['Wroot']
        smax = jax.ops.segment_max(xc, batch, num_segments=N_GRAPHS_K)
        ex = jnp.exp(xc - smax[batch])
        den = jax.ops.segment_sum(ex, batch, num_segments=N_GRAPHS_K)
        scores = ex / den[batch]
        gx = jax.ops.segment_sum(out * scores, batch, num_segments=N_GRAPHS_K)
        out_list.append(out)
        gout_list.append(jnp.tanh(gx @ params['Wgout'] + params['cgout']))
    gout_all = jnp.stack(gout_list, axis=-1)
    out_all = jnp.stack(out_list, axis=-1)
    sc = jnp.sum(gout_all * params['a'], axis=1, keepdims=True) + params['a_bias']
    sc = jax.nn.softmax(sc, axis=-1)
    sc_e = sc[batch]
    out = jnp.sum(out_all * sc_e, axis=-1)
    node_agg = jax.ops.segment_sum(out, edge_index[1], num_segments=N_NODES_K)
    h = x + node_agg
    h = pl.pallas_call(
        _copy_kernel,
        out_shape=jax.ShapeDtypeStruct(h.shape, h.dtype),
    )(h)
    return _linear_block_k(h, params)


# speedup vs baseline: 2.4676x; 2.4676x over previous
import functools

import jax
import jax.numpy as jnp
from jax import lax
from jax.experimental import pallas as pl
from jax.experimental.pallas import tpu as pltpu
from jax.experimental.pallas import tpu_sc as plsc

N = 10000
E = 320000
L = 640000
G = 128
D = 128
NIT = 3
SND = 6 * D

NC = 2
NS = 16
NWK = NC * NS

CH = 8192
NB = 40
ACC_ROWS = 16 * 513
WPAIRS = L // NWK
BIN_CAP = L + NWK * NB * 8

@functools.cache
def _mesh():
    return plsc.VectorSubcoreMesh(core_axis_name="c", subcore_axis_name="s",
                                  num_cores=NC, num_subcores=NS)


def _sc_kernel(body, out_type, scratch_types):
    built = []

    def call(*args):
        if not built:
            built.append(pl.kernel(
                body, out_type=out_type, mesh=_mesh(),
                scratch_types=scratch_types,
                compiler_params=pltpu.CompilerParams(
                    needs_layout_passes=False)))
        return built[0](*args)

    return call


def _sds(shape, dtype=jnp.float32):
    return jax.ShapeDtypeStruct(shape, dtype)


def _sload(ref, i):
    return ref[pl.ds(i, 16)][0]


def _rank_and_last(binsbuf, pos0, b, iota):
    r = jnp.zeros((16,), jnp.int32)
    later = jnp.zeros((16,), jnp.int32)
    for k in range(1, 16):
        pv = plsc.load_gather(binsbuf, [pos0 + jnp.maximum(iota - k, 0)])
        r = r + jnp.where((iota >= k) & (pv == b), 1, 0)
        nv = plsc.load_gather(binsbuf, [pos0 + jnp.minimum(iota + k, 15)])
        later = later + jnp.where((iota < 16 - k) & (nv == b), 1, 0)
    return r, later == 0


def _mm_body(x_ref, w_ref, o_ref):
    o_ref[...] = jnp.dot(x_ref[...], w_ref[...],
                         preferred_element_type=jnp.float32)


def _matmul(x, w, block_rows):
    m, k = x.shape
    _, n = w.shape
    grid = m // block_rows
    return pl.pallas_call(
        _mm_body,
        grid=(grid,),
        in_specs=[
            pl.BlockSpec((block_rows, k), lambda i: (i, 0)),
            pl.BlockSpec((k, n), lambda i: (0, 0)),
        ],
        out_specs=pl.BlockSpec((block_rows, n), lambda i: (i, 0)),
        out_shape=_sds((m, n)),
    )(x, w)


_EA_K = 512
_EA_CHUNKS = E // _EA_K


def _ea_body(eu3, ev3, euv3, ei0, ei1, aranges, ea_out,
             idx0, idx1, idxl, rows, sem):
    wid = lax.axis_index("s") * NC + lax.axis_index("c")

    def body(j, _):
        cid = j * NWK + wid

        @pl.when(cid < _EA_CHUNKS)
        def _():
            base = cid * _EA_K
            pltpu.sync_copy(ei0.at[pl.ds(base, _EA_K)], idx0)
            pltpu.sync_copy(ei1.at[pl.ds(base, _EA_K)], idx1)
            pltpu.sync_copy(aranges.at[pl.ds(base, _EA_K)], idxl)
            pltpu.async_copy(eu3.at[idx0], rows, sem).wait()
            pltpu.async_copy(ev3.at[idx1], rows, sem, add=True).wait()
            pltpu.async_copy(euv3.at[idxl], rows, sem, add=True).wait()
            pltpu.sync_copy(rows, ea_out.at[pl.ds(base, _EA_K)])
        return 0

    lax.fori_loop(0, (_EA_CHUNKS + NWK - 1) // NWK, body, 0)


_ea_kernel = _sc_kernel(
    _ea_body,
    _sds((E, D)),
    [
        pltpu.VMEM((_EA_K,), jnp.int32),
        pltpu.VMEM((_EA_K,), jnp.int32),
        pltpu.VMEM((_EA_K,), jnp.int32),
        pltpu.VMEM((_EA_K, D), jnp.float32),
        pltpu.SemaphoreType.DMA,
    ])


_CNT_CHUNK = 2000
_CNT_PAD = 48 * NWK


def _bin_count_body(lgdst, counts_out, dbuf, bbuf, hist, hidx, sem):
    wid = lax.axis_index("s") * NC + lax.axis_index("c")
    iota = lax.iota(jnp.int32, 16)

    def zero_hist(i, _):
        hist[pl.ds(i * 16, 16)] = jnp.zeros((16,), jnp.int32)
        return 0

    lax.fori_loop(0, 3, zero_hist, 0)

    def chunk_body(j, _):
        base = wid * WPAIRS + j * _CNT_CHUNK
        pltpu.sync_copy(lgdst.at[pl.ds(base, _CNT_CHUNK)], dbuf)

        def binify(i, _):
            bbuf[pl.ds(i * 16, 16)] = dbuf[pl.ds(i * 16, 16)] >> 13
            return 0

        lax.fori_loop(0, _CNT_CHUNK // 16, binify, 0)

        def vreg_body(i, _):
            b = bbuf[pl.ds(i * 16, 16)]
            r, last = _rank_and_last(bbuf, i * 16, b, iota)
            plsc.addupdate_scatter(hist, [b], r + 1, mask=last)
            return 0

        lax.fori_loop(0, _CNT_CHUNK // 16, vreg_body, 0)
        return 0

    lax.fori_loop(0, WPAIRS // _CNT_CHUNK, chunk_body, 0)

    def mk_idx(i, _):
        hidx[pl.ds(i * 16, 16)] = (i * 16 + iota) * NWK + wid
        return 0

    lax.fori_loop(0, 3, mk_idx, 0)
    pltpu.async_copy(hist, counts_out.at[hidx], sem).wait()


_bin_count_kernel = _sc_kernel(
    _bin_count_body,
    _sds((_CNT_PAD,), jnp.int32),
    [
        pltpu.VMEM((_CNT_CHUNK,), jnp.int32),
        pltpu.VMEM((_CNT_CHUNK,), jnp.int32),
        pltpu.VMEM((48,), jnp.int32),
        pltpu.VMEM((48,), jnp.int32),
        pltpu.SemaphoreType.DMA,
    ])


_LOC_CAP = WPAIRS + NB * 8 + 16


def _bin_compact_body(lgsrc, lgdst, counts, binned_out, binstart_out,
                      sbuf, dbuf, bbuf, pkstage, posstage, sentstage,
                      sposstage, cnt_v, offs_v, base_v, bs_v,
                      cur_s, cnt_s, goff_s, sem):
    core = lax.axis_index("c")
    tile = lax.axis_index("s")
    wid = tile * NC + core
    iota = lax.iota(jnp.int32, 16)

    def stage(j, _):
        base = wid * WPAIRS + j * _CNT_CHUNK
        pltpu.sync_copy(lgsrc.at[pl.ds(base, _CNT_CHUNK)],
                        sbuf.at[pl.ds(j * _CNT_CHUNK, _CNT_CHUNK)])
        pltpu.sync_copy(lgdst.at[pl.ds(base, _CNT_CHUNK)],
                        dbuf.at[pl.ds(j * _CNT_CHUNK, _CNT_CHUNK)])
        return 0

    lax.fori_loop(0, WPAIRS // _CNT_CHUNK, stage, 0)

    pltpu.sync_copy(counts.at[pl.ds(0, NB * NWK)], cnt_v.at[pl.ds(0, NB * NWK)])

    def scan_body(i, carry):
        c = cnt_v[pl.ds(i * 16, 16)]
        c8 = (c + 7) & jnp.full((16,), ~7, jnp.int32)
        incl = plsc.cumsum(c8)
        offs_v[pl.ds(i * 16, 16)] = incl - c8 + carry
        return carry + jnp.max(incl)

    total = lax.fori_loop(0, NB * NWK // 16, scan_body, 0)

    for i in range(3):
        idx = (i * 16 + iota) * NWK + wid
        c = plsc.load_gather(cnt_v, [idx])
        gofv = plsc.load_gather(offs_v, [idx])
        for j in range(16):
            b = i * 16 + j
            if b < NB:
                cur_s[b] = gofv[j]
                cnt_s[b] = c[j]
                goff_s[b] = gofv[j]

    def binify(i, _):
        bbuf[pl.ds(i * 16, 16)] = dbuf[pl.ds(i * 16, 16)] >> 13
        return 0

    lax.fori_loop(0, WPAIRS // 16, binify, 0)

    def place_body(i, _):
        for v in range(8):
            off = (i * 8 + v) * 16
            d = dbuf[pl.ds(off, 16)]
            s = sbuf[pl.ds(off, 16)]
            bv = bbuf[pl.ds(off, 16)]
            dloc = d & jnp.full((16,), CH - 1, jnp.int32)
            pk = s | (dloc << 19)
            pos = jnp.zeros((16,), jnp.int32)
            for j in range(16):
                bj = bv[j]
                cj = cur_s[bj]
                cur_s[bj] = cj + 1
                pos = jnp.where(iota == j, cj, pos)
            pkstage[pl.ds(v * 16, 16)] = pk
            posstage[pl.ds(v * 16, 16)] = pos
        pltpu.async_copy(pkstage, binned_out.at[posstage], sem).wait()
        return 0

    lax.fori_loop(0, WPAIRS // 128, place_body, 0)

    sent = ((jnp.full((16,), CH, jnp.int32) + (iota & 7)) << 19) | wid

    def pad_flush(b, _):
        cnt = cnt_s[b]
        c8 = (cnt + 7) & ~7
        fill = cur_s[b]
        npad = c8 - cnt
        sentstage[...] = sent
        sposstage[...] = jnp.where(iota < npad, fill + iota, BIN_CAP + iota)
        pltpu.async_copy(sentstage, binned_out.at[sposstage], sem).wait()
        return 0

    lax.fori_loop(0, NB, pad_flush, 0)

    @pl.when(wid == 0)
    def _():
        offs_v[pl.ds(NB * NWK, 16)] = jnp.broadcast_to(total, (16,))

        def bs_body(i, _):
            idx = jnp.minimum((i * 16 + iota) * NWK, NB * NWK)
            bs_v[pl.ds(i * 16, 16)] = plsc.load_gather(offs_v, [idx])
            return 0

        lax.fori_loop(0, 3, bs_body, 0)
        pltpu.sync_copy(bs_v, binstart_out)


_bin_compact_kernel = _sc_kernel(
    _bin_compact_body,
    (_sds((BIN_CAP + 48,), jnp.int32), _sds((48,), jnp.int32)),
    [
        pltpu.VMEM((WPAIRS,), jnp.int32),
        pltpu.VMEM((WPAIRS,), jnp.int32),
        pltpu.VMEM((WPAIRS,), jnp.int32),
        pltpu.VMEM((128,), jnp.int32),
        pltpu.VMEM((128,), jnp.int32),
        pltpu.VMEM((16,), jnp.int32),
        pltpu.VMEM((16,), jnp.int32),
        pltpu.VMEM((NB * NWK + 16,), jnp.int32),
        pltpu.VMEM((NB * NWK + 16,), jnp.int32),
        pltpu.VMEM((64,), jnp.int32),
        pltpu.VMEM((48,), jnp.int32),
        pltpu.SMEM((NB,), jnp.int32),
        pltpu.SMEM((NB,), jnp.int32),
        pltpu.SMEM((NB,), jnp.int32),
        pltpu.SemaphoreType.DMA,
    ])


_SEG_K = 128


def _make_seg_kernel(is_agg):
    scratch = [
        pltpu.VMEM_SHARED((ACC_ROWS, D), jnp.float32),
        pltpu.VMEM((128, D), jnp.float32),
        pltpu.VMEM((64,), jnp.int32),
        pltpu.VMEM((_SEG_K,), jnp.int32),
        pltpu.VMEM((_SEG_K,), jnp.int32),
        pltpu.VMEM((_SEG_K,), jnp.int32),
        pltpu.VMEM((_SEG_K, D), jnp.float32),
        pltpu.VMEM((8,), jnp.int32),
        pltpu.VMEM((8,), jnp.int32),
        pltpu.VMEM((8,), jnp.int32),
        pltpu.VMEM((8, D), jnp.float32),
        pltpu.VMEM((128, D), jnp.float32),
        pltpu.VMEM((128,), jnp.int32),
        pltpu.SMEM((48,), jnp.int32),
        pltpu.SemaphoreType.DMA,
    ]

    def body(src_data, binned, binstart, ea, aranges, out_ref,
             acc, zbuf, bs_v, pkb, srb, dlb, rows, pk8, sr8, dl8, rows8,
             fbuf, fidx, bs_s, sem):
        core = lax.axis_index("c")
        tile = lax.axis_index("s")

        pltpu.sync_copy(binstart, bs_v.at[pl.ds(0, 48)])
        for i in range(3):
            v = bs_v[pl.ds(i * 16, 16)]
            for j in range(16):
                bs_s[i * 16 + j] = v[j]

        def zb(i, _):
            zbuf[lax.shift_right_logical(i, 3),
                 pl.ds(lax.bitwise_and(i, 7) * 16, 16)] = (
                     jnp.zeros((16,), jnp.float32))
            return 0

        lax.fori_loop(0, 128 * (D // 16), zb, 0)

        def process(base, k, pk_r, sr_r, dl_r, row_r):
            base = pl.multiple_of(base, 8)
            pltpu.sync_copy(binned.at[pl.ds(base, k)], pk_r)

            def unp(i, _):
                v = pk_r[pl.ds(i * 16, 16)]
                sr_r[pl.ds(i * 16, 16)] = lax.bitwise_and(
                    v, jnp.full((16,), 0x7FFFF, jnp.int32))
                dl_r[pl.ds(i * 16, 16)] = lax.shift_right_logical(
                    v, jnp.full((16,), 19, jnp.int32))
                return 0

            lax.fori_loop(0, k // 16, unp, 0)
            pltpu.async_copy(src_data.at[sr_r], row_r, sem).wait()
            pltpu.sync_copy(row_r, acc.at[dl_r], add=True)

        def chunk_body(ci, _):
            c = ci * NC + core

            def z4(k, _):
                pltpu.sync_copy(zbuf, acc.at[pl.ds(tile * 513 + k * 128, 128)])
                return 0

            lax.fori_loop(0, 4, z4, 0)
            pltpu.sync_copy(zbuf.at[pl.ds(0, 1)],
                            acc.at[pl.ds(tile * 513 + 512, 1)])
            plsc.subcore_barrier()

            lo = bs_s[c]
            hi = bs_s[c + 1]
            ln = hi - lo
            a0 = pl.multiple_of(
                lo + lax.bitwise_and(
                    lax.shift_right_logical(ln * tile, 4), ~7), 8)
            a1 = jnp.where(tile == NS - 1,
                           hi,
                           lo + lax.bitwise_and(
                               lax.shift_right_logical(ln * (tile + 1), 4), ~7))
            n = a1 - a0
            n128 = lax.shift_right_logical(n, 7)

            def big(j, _):
                process(a0 + j * _SEG_K, _SEG_K, pkb, srb, dlb, rows)
                return 0

            lax.fori_loop(0, n128, big, 0)
            done = n128 * _SEG_K

            def small(j, _):
                process(a0 + done + j * 8, 8, pk8, sr8, dl8, rows8)
                return 0

            lax.fori_loop(0, lax.shift_right_logical(n - done, 3), small, 0)
            plsc.subcore_barrier()

            g0 = c * CH

            def flush(rbase, nrows, fb, fi):
                pltpu.sync_copy(acc.at[pl.ds(rbase, nrows)],
                                fb.at[pl.ds(0, nrows)])
                if is_agg:
                    pltpu.sync_copy(aranges.at[pl.ds(g0 + rbase, nrows)],
                                    fi.at[pl.ds(0, nrows)])
                    pltpu.async_copy(ea.at[fi.at[pl.ds(0, nrows)]],
                                     fb.at[pl.ds(0, nrows)], sem,
                                     add=True).wait()
                pltpu.sync_copy(fb.at[pl.ds(0, nrows)],
                                out_ref.at[pl.ds(g0 + rbase, nrows)])

            @pl.when(c < NB - 1)
            def _():
                def f4(k, _):
                    flush(tile * 512 + k * 128, 128, fbuf, fidx)
                    return 0

                lax.fori_loop(0, 4, f4, 0)

            @pl.when(c == NB - 1)
            def _():
                flush(tile * 32, 32, fbuf, fidx)

            plsc.subcore_barrier()
            return 0

        lax.fori_loop(0, NB // NC, chunk_body, 0)

    return _sc_kernel(body, _sds((E, D)), scratch)


_seg_agg = _make_seg_kernel(True)
_seg_conv = _make_seg_kernel(False)


_NA_K = 128
_NA_PER_TILE = E // NWK


def _node_agg_body(out_final, ei1, nacc_out, acc, zbuf, idx, idx_r, rows,
                   sem):
    core = lax.axis_index("c")
    tile = lax.axis_index("s")

    def zb(i, _):
        zbuf[i // 8, pl.ds((i % 8) * 16, 16)] = jnp.zeros((16,), jnp.float32)
        return 0

    lax.fori_loop(0, 128 * (D // 16), zb, 0)

    def z4(k, _):
        pltpu.sync_copy(zbuf, acc.at[pl.ds(tile * 624 + k * 128, 128)])
        return 0

    lax.fori_loop(0, 4, z4, 0)

    @pl.when(tile < NS - 1)
    def _():
        pltpu.sync_copy(zbuf.at[pl.ds(0, 112)],
                        acc.at[pl.ds(tile * 624 + 512, 112)])

    @pl.when(tile == NS - 1)
    def _():
        pltpu.sync_copy(zbuf, acc.at[pl.ds(15 * 624 + 512, 128)])
    plsc.subcore_barrier()

    ebase = core * (E // NC) + tile * _NA_PER_TILE

    def chunk(j, _):
        base = ebase + j * _NA_K
        pltpu.sync_copy(ei1.at[pl.ds(base, _NA_K)], idx)
        pltpu.sync_copy(out_final.at[pl.ds(base, _NA_K)], rows)
        pltpu.sync_copy(rows, acc.at[idx], add=True)
        return 0

    lax.fori_loop(0, _NA_PER_TILE // _NA_K, chunk, 0)
    rem = _NA_PER_TILE - (_NA_PER_TILE // _NA_K) * _NA_K
    base = ebase + (_NA_PER_TILE // _NA_K) * _NA_K
    pltpu.sync_copy(ei1.at[pl.ds(base, rem)], idx_r)
    pltpu.sync_copy(out_final.at[pl.ds(base, rem)], rows.at[pl.ds(0, rem)])
    pltpu.sync_copy(rows.at[pl.ds(0, rem)], acc.at[idx_r], add=True)
    plsc.subcore_barrier()

    def fl(k, _):
        pltpu.sync_copy(acc.at[pl.ds(tile * 624 + k * 128, 128)], zbuf)
        pltpu.sync_copy(zbuf,
                        nacc_out.at[core, pl.ds(tile * 624 + k * 128, 128)])
        return 0

    lax.fori_loop(0, 4, fl, 0)

    @pl.when(tile < NS - 1)
    def _():
        pltpu.sync_copy(acc.at[pl.ds(tile * 624 + 512, 112)],
                        zbuf.at[pl.ds(0, 112)])
        pltpu.sync_copy(zbuf.at[pl.ds(0, 112)],
                        nacc_out.at[core, pl.ds(tile * 624 + 512, 112)])

    @pl.when(tile == NS - 1)
    def _():
        pltpu.sync_copy(acc.at[pl.ds(15 * 624 + 512, 128)], zbuf)
        pltpu.sync_copy(zbuf, nacc_out.at[core, pl.ds(15 * 624 + 512, 128)])


_node_agg_kernel = _sc_kernel(
    _node_agg_body,
    _sds((NC, N, D)),
    [
        pltpu.VMEM_SHARED((N, D), jnp.float32),
        pltpu.VMEM((128, D), jnp.float32),
        pltpu.VMEM((_NA_K,), jnp.int32),
        pltpu.VMEM((16,), jnp.int32),
        pltpu.VMEM((_NA_K, D), jnp.float32),
        pltpu.SemaphoreType.DMA,
    ])


_BE = 2560
_NEB = E // _BE


def _p1_body(conv_ref, out_ref_, b_ref, wrel_ref, wroot_ref, crel_ref,
             xc_ref, smax_ref):
    j = pl.program_id(0)
    xc = (jnp.dot(conv_ref[...], wrel_ref[...],
                  preferred_element_type=jnp.float32)
          + jnp.dot(out_ref_[...], wroot_ref[...],
                    preferred_element_type=jnp.float32)
          + crel_ref[0, 0])
    xc_ref[...] = xc
    mask = b_ref[...] == lax.broadcasted_iota(jnp.int32, (1, G), 1)
    m = jnp.max(jnp.where(mask, xc, -1e30), axis=0, keepdims=True)

    @pl.when(j == 0)
    def _():
        smax_ref[...] = jnp.full((1, G), -1e30, jnp.float32)

    smax_ref[...] = jnp.maximum(smax_ref[...], m)


def _p1(conv, out, batch2d, wrel, wroot, crel):
    return pl.pallas_call(
        _p1_body,
        grid=(_NEB,),
        in_specs=[
            pl.BlockSpec((_BE, D), lambda j: (j, 0)),
            pl.BlockSpec((_BE, D), lambda j: (j, 0)),
            pl.BlockSpec((_BE, 1), lambda j: (j, 0)),
            pl.BlockSpec((D, 1), lambda j: (0, 0)),
            pl.BlockSpec((D, 1), lambda j: (0, 0)),
            pl.BlockSpec((1, 1), lambda j: (0, 0)),
        ],
        out_specs=[
            pl.BlockSpec((_BE, 1), lambda j: (j, 0)),
            pl.BlockSpec((1, G), lambda j: (0, 0)),
        ],
        out_shape=[_sds((E, 1)), _sds((1, G))],
        compiler_params=pltpu.CompilerParams(
            dimension_semantics=("arbitrary",)),
    )(conv, out, batch2d, wrel, wroot, crel)


def _p2_body(out_ref_, xc_ref, b_ref, smax_ref, den_ref, gxn_ref):
    j = pl.program_id(0)
    mask = (b_ref[...] == lax.broadcasted_iota(jnp.int32, (1, G), 1)).astype(
        jnp.float32)
    smax_e = jnp.sum(mask * smax_ref[...], axis=1, keepdims=True)
    ex = jnp.exp(xc_ref[...] - smax_e)
    mex = mask * ex

    @pl.when(j == 0)
    def _():
        den_ref[...] = jnp.zeros((1, G), jnp.float32)
        gxn_ref[...] = jnp.zeros((G, G), jnp.float32)

    den_ref[...] += jnp.sum(mex, axis=0, keepdims=True)
    gxn_ref[...] += lax.dot_general(mex, out_ref_[...],
                                    (((0,), (0,)), ((), ())),
                                    preferred_element_type=jnp.float32)


def _p2(out, xc, batch2d, smax):
    return pl.pallas_call(
        _p2_body,
        grid=(_NEB,),
        in_specs=[
            pl.BlockSpec((_BE, D), lambda j: (j, 0)),
            pl.BlockSpec((_BE, 1), lambda j: (j, 0)),
            pl.BlockSpec((_BE, 1), lambda j: (j, 0)),
            pl.BlockSpec((1, G), lambda j: (0, 0)),
        ],
        out_specs=[
            pl.BlockSpec((1, G), lambda j: (0, 0)),
            pl.BlockSpec((G, G), lambda j: (0, 0)),
        ],
        out_shape=[_sds((1, G)), _sds((G, G))],
        compiler_params=pltpu.CompilerParams(
            dimension_semantics=("arbitrary",)),
    )(out, xc, batch2d, smax)


def _p3_body(gxn_ref, den_ref, wg_ref, cg_ref, a_ref, ab_ref, sc_ref):
    cols = []
    for i in range(NIT):
        den = jnp.maximum(den_ref[i, :].reshape(G, 1), 1e-30)
        gx = gxn_ref[i, :, :] / den
        gout = jnp.tanh(jnp.dot(gx, wg_ref[...],
                                preferred_element_type=jnp.float32)
                        + cg_ref[...])
        s_i = jnp.sum(gout * a_ref[:, i].reshape(1, D), axis=1,
                      keepdims=True) + ab_ref[0, i]
        cols.append(s_i)
    s = jnp.concatenate(cols, axis=1)
    m = jnp.max(s, axis=1, keepdims=True)
    e = jnp.exp(s - m)
    sc_ref[...] = e / jnp.sum(e, axis=1, keepdims=True)


def _p3(gxn_all, den_all, wgout, cgout, a2, ab2):
    return pl.pallas_call(
        _p3_body,
        in_specs=[
            pl.BlockSpec(s.shape, lambda idx=tuple([0] * len(s.shape)): idx)
            for s in (gxn_all, den_all, wgout, cgout, a2, ab2)],
        out_specs=pl.BlockSpec((G, NIT), lambda: (0, 0)),
        out_shape=_sds((G, NIT)),
    )(gxn_all, den_all, wgout, cgout, a2, ab2)


def _combine_body(o1_ref, o2_ref, o3_ref, b_ref, sc_ref, of_ref):
    mask = (b_ref[...] == lax.broadcasted_iota(jnp.int32, (1, G), 1)).astype(
        jnp.float32)
    w = jnp.dot(mask, sc_ref[...], preferred_element_type=jnp.float32)
    of_ref[...] = (w[:, 0:1] * o1_ref[...] + w[:, 1:2] * o2_ref[...]
                   + w[:, 2:3] * o3_ref[...])


def _combine(o1, o2, o3, batch2d, sc):
    return pl.pallas_call(
        _combine_body,
        grid=(_NEB,),
        in_specs=[
            pl.BlockSpec((_BE, D), lambda j: (j, 0)),
            pl.BlockSpec((_BE, D), lambda j: (j, 0)),
            pl.BlockSpec((_BE, D), lambda j: (j, 0)),
            pl.BlockSpec((_BE, 1), lambda j: (j, 0)),
            pl.BlockSpec((G, NIT), lambda j: (0, 0)),
        ],
        out_specs=pl.BlockSpec((_BE, D), lambda j: (j, 0)),
        out_shape=_sds((E, D)),
    )(o1, o2, o3, batch2d, sc)


_BN_BLK = 2000
_NBLK = N // _BN_BLK


def _stats_epilogue(j, y, sum_ref, sq_ref, mean_ref, rstd_ref):
    @pl.when(j == 0)
    def _():
        sum_ref[...] = jnp.zeros_like(sum_ref)
        sq_ref[...] = jnp.zeros_like(sq_ref)

    sum_ref[...] += jnp.sum(y, axis=0, keepdims=True)
    sq_ref[...] += jnp.sum(y * y, axis=0, keepdims=True)

    @pl.when(j == _NBLK - 1)
    def _():
        mean = sum_ref[...] / N
        var = sq_ref[...] / N - mean * mean
        mean_ref[...] = mean
        rstd_ref[...] = lax.rsqrt(var + 1e-5)


def _s0_body(x_ref, n0_ref, n1_ref, h_ref, mean_ref, rstd_ref,
             sum_ref, sq_ref):
    j = pl.program_id(0)
    h = x_ref[...] + n0_ref[...] + n1_ref[...]
    h_ref[...] = h
    _stats_epilogue(j, h, sum_ref, sq_ref, mean_ref, rstd_ref)


def _s0(x, n0, n1):
    return pl.pallas_call(
        _s0_body,
        grid=(_NBLK,),
        in_specs=[pl.BlockSpec((_BN_BLK, D), lambda j: (j, 0))] * 3,
        out_specs=[
            pl.BlockSpec((_BN_BLK, D), lambda j: (j, 0)),
            pl.BlockSpec((1, D), lambda j: (0, 0)),
            pl.BlockSpec((1, D), lambda j: (0, 0)),
        ],
        out_shape=[_sds((N, D)), _sds((1, D)), _sds((1, D))],
        scratch_shapes=[pltpu.VMEM((1, D), jnp.float32)] * 2,
        compiler_params=pltpu.CompilerParams(
            dimension_semantics=("arbitrary",)),
    )(x, n0, n1)


def _make_stage(cin, cout, prelu, resid, stats):
    def body(*refs):
        i = 0
        x_ref = refs[i]; i += 1
        if resid:
            r_ref = refs[i]; i += 1
        mean_ref_in = refs[i]; i += 1
        rstd_ref_in = refs[i]; i += 1
        g_ref = refs[i]; i += 1
        b_ref = refs[i]; i += 1
        if prelu:
            p_ref = refs[i]; i += 1
        w_ref = refs[i]; i += 1
        c_ref = refs[i]; i += 1
        y_ref = refs[i]; i += 1
        if stats:
            mean_ref, rstd_ref, sum_ref, sq_ref = refs[i:i + 4]
        j = pl.program_id(0)
        xb = (x_ref[...] - mean_ref_in[...]) * rstd_ref_in[...]
        xb = xb * g_ref[...] + b_ref[...]
        if prelu:
            p = p_ref[0, 0]
            xb = jnp.where(xb >= 0, xb, p * xb)
        y = jnp.dot(xb, w_ref[...], preferred_element_type=jnp.float32)
        y = y + c_ref[...]
        if resid:
            y = (y + r_ref[...]) * 0.5
        y_ref[...] = y
        if stats:
            _stats_epilogue(j, y, sum_ref, sq_ref, mean_ref, rstd_ref)

    def call(x, resid_x, mean, rstd, g, b, p, w, c):
        in_arrs = [x]
        in_specs = [pl.BlockSpec((_BN_BLK, cin), lambda j: (j, 0))]
        if resid:
            in_arrs.append(resid_x)
            in_specs.append(pl.BlockSpec((_BN_BLK, cout), lambda j: (j, 0)))
        in_arrs += [mean, rstd, g, b]
        in_specs += [pl.BlockSpec((1, cin), lambda j: (0, 0))] * 4
        if prelu:
            in_arrs.append(p)
            in_specs.append(pl.BlockSpec((1, 1), lambda j: (0, 0)))
        in_arrs += [w, c]
        in_specs += [pl.BlockSpec((cin, cout), lambda j: (0, 0)),
                     pl.BlockSpec((1, cout), lambda j: (0, 0))]
        out_specs = [pl.BlockSpec((_BN_BLK, cout), lambda j: (j, 0))]
        out_shape = [_sds((N, cout))]
        if stats:
            out_specs += [pl.BlockSpec((1, cout), lambda j: (0, 0))] * 2
            out_shape += [_sds((1, cout))] * 2
        res = pl.pallas_call(
            body,
            grid=(_NBLK,),
            in_specs=in_specs,
            out_specs=out_specs,
            out_shape=out_shape,
            scratch_shapes=([pltpu.VMEM((1, cout), jnp.float32)] * 2
                            if stats else []),
            compiler_params=pltpu.CompilerParams(
                dimension_semantics=("arbitrary",)),
        )(*in_arrs)
        return res if stats else res[0]

    return call


_stage1 = _make_stage(D, SND, prelu=False, resid=False, stats=True)
_stage2 = _make_stage(SND, SND, prelu=True, resid=False, stats=True)
_stage3 = _make_stage(SND, SND, prelu=True, resid=True, stats=True)
_stage4 = _make_stage(SND, SND, prelu=True, resid=True, stats=True)
_stage5 = _make_stage(SND, D, prelu=True, resid=False, stats=False)


def kernel(x, edge_attr, params, edge_index, line_graph_edge_index,
           edge_index_batch):
    p = params
    ei0 = edge_index[0]
    ei1 = edge_index[1]
    lgsrc = line_graph_edge_index[0]
    lgdst = line_graph_edge_index[1]
    batch2d = edge_index_batch.reshape(E, 1)
    aranges = jnp.arange(E, dtype=jnp.int32)

    wuv3 = jnp.concatenate([p['Wu'], p['Wv']], axis=1) / 3.0
    z = _matmul(x, wuv3, 2000)
    eu3 = z[:, :D]
    ev3 = z[:, D:]
    euv3 = _matmul(edge_attr, p['We'] / 3.0, _BE)

    ea = _ea_kernel(eu3, ev3, euv3, ei0, ei1, aranges)

    wrel = p['Wrel']
    wroot = p['Wroot']
    crel = p['crel'].reshape(1, 1)

    outs = []
    dens = []
    gxns = []
    src = ea
    for _ in range(NIT):
        out_i = ea + jax.ops.segment_sum(src[lgsrc], lgdst, num_segments=E)
        conv_i = jax.ops.segment_sum(out_i[lgsrc], lgdst, num_segments=E)
        xc, smax = _p1(conv_i, out_i, batch2d, wrel, wroot, crel)
        den, gxn = _p2(out_i, xc, batch2d, smax)
        outs.append(out_i)
        dens.append(den)
        gxns.append(gxn)
        src = out_i

    gxn_all = jnp.stack(gxns, axis=0)
    den_all = jnp.concatenate(dens, axis=0)
    sc = _p3(gxn_all, den_all, p['Wgout'], p['cgout'].reshape(1, D),
             p['a'][0], p['a_bias'][0])

    out_final = _combine(outs[0], outs[1], outs[2], batch2d, sc)
    nacc = _node_agg_kernel(out_final, ei1)

    h, mean0, rstd0 = _s0(x, nacc[0], nacc[1])
    x1, mean1, rstd1 = _stage1(h, None, mean0, rstd0,
                               p['g1'].reshape(1, D), p['b1'].reshape(1, D),
                               None, p['W1'], p['c1'].reshape(1, SND))
    x2, mean2, rstd2 = _stage2(x1, None, mean1, rstd1,
                               p['g2'].reshape(1, SND), p['b2'].reshape(1, SND),
                               p['p2'].reshape(1, 1), p['W2'],
                               p['c2'].reshape(1, SND))
    xa, meana, rstda = _stage3(x2, x1, mean2, rstd2,
                               p['g3'].reshape(1, SND), p['b3'].reshape(1, SND),
                               p['p3'].reshape(1, 1), p['W3'],
                               p['c3'].reshape(1, SND))
    xb, meanb, rstdb = _stage4(xa, xa, meana, rstda,
                               p['g4'].reshape(1, SND), p['b4'].reshape(1, SND),
                               p['p4'].reshape(1, 1), p['W4'],
                               p['c4'].reshape(1, SND))
    y = _stage5(xb, None, meanb, rstdb,
                p['g5'].reshape(1, SND), p['b5'].reshape(1, SND),
                p['p5'].reshape(1, 1), p['W5'], p['c5'].reshape(1, D))
    return y
